# baseline (speedup 1.0000x reference)
import functools

import jax
import jax.numpy as jnp
from jax import lax
from jax.experimental import pallas as pl
from jax.experimental.pallas import tpu as pltpu
from jax.experimental.pallas import tpu_sc as plsc

NC = 2
NS = 16
NW = NC * NS
CHUNK = 64
NBUF = 2


def _gather_body(uidx_hbm, iidx_hbm, t_umlp, t_imlp, t_umf, t_imf,
                 o_umlp, o_imlp, o_umf, o_imf,
                 uidx_v, iidx_v, urow_v, irow_v, bufs, sems,
                 *, b_per_w):
    wid = lax.axis_index("s") * NC + lax.axis_index("c")
    base = wid * b_per_w
    pltpu.sync_copy(uidx_hbm.at[pl.ds(base, b_per_w)], uidx_v)
    pltpu.sync_copy(iidx_hbm.at[pl.ds(base, b_per_w)], iidx_v)
    for i in range(b_per_w // 16):
        sl = pl.ds(i * 16, 16)
        urow_v[sl] = lax.shift_right_logical(uidx_v[sl], 3)
        irow_v[sl] = lax.shift_right_logical(iidx_v[sl], 3)

    tables = (t_umlp, t_imlp, t_umf, t_imf)
    outs = (o_umlp, o_imlp, o_umf, o_imf)
    rows = (urow_v, irow_v, urow_v, irow_v)
    n_chunks = b_per_w // CHUNK

    def issue(c, slot):
        cs = pl.ds(c * CHUNK, CHUNK)
        return [pltpu.async_copy(t.at[r.at[cs]], bufs[slot][k], sems[slot])
                for k, (t, r) in enumerate(zip(tables, rows))]

    def drain(handles, c, slot):
        cs = pl.ds(base + c * CHUNK, CHUNK)
        for k, h in enumerate(handles):
            h.wait()
        wr = [pltpu.async_copy(bufs[slot][k], outs[k].at[cs], sems[slot])
              for k in range(4)]
        return wr

    pending_wr = [None] * NBUF
    pending_rd = [None] * NBUF
    for c in range(n_chunks + 1):
        slot = c % NBUF
        if c < n_chunks:
            if pending_wr[slot] is not None:
                for h in pending_wr[slot]:
                    h.wait()
                pending_wr[slot] = None
            pending_rd[slot] = issue(c, slot)
        if c >= 1:
            pslot = (c - 1) % NBUF
            pending_wr[pslot] = drain(pending_rd[pslot], c - 1, pslot)
            pending_rd[pslot] = None
    for slot in range(NBUF):
        if pending_wr[slot] is not None:
            for h in pending_wr[slot]:
                h.wait()


@functools.lru_cache(maxsize=None)
def _make_gather(B):
    assert B % (8 * NW) == 0
    b_per_w = B // NW
    assert b_per_w % CHUNK == 0
    mesh = plsc.VectorSubcoreMesh(core_axis_name="c", subcore_axis_name="s",
                                  num_cores=NC, num_subcores=NS)
    f32 = jnp.float32
    out = jax.ShapeDtypeStruct((B, 128), f32)
    return pl.kernel(
        functools.partial(_gather_body, b_per_w=b_per_w),
        out_type=(out, out, out, out),
        mesh=mesh,
        scratch_types=[
            pltpu.VMEM((b_per_w,), jnp.int32),
            pltpu.VMEM((b_per_w,), jnp.int32),
            pltpu.VMEM((b_per_w,), jnp.int32),
            pltpu.VMEM((b_per_w,), jnp.int32),
            [[pltpu.VMEM((CHUNK, 128), f32) for _ in range(4)]
             for _ in range(NBUF)],
            [pltpu.SemaphoreType.DMA for _ in range(NBUF)],
        ],
    )


def _mlp_body(uidx_ref, iidx_ref, ru_mlp_ref, ri_mlp_ref, ru_mf_ref, ri_mf_ref,
              W1_ref, b1_ref, W2_ref, b2_ref, W3_ref, b3_ref, Wa_ref, ba_ref,
              out_ref):
    blk = uidx_ref.shape[0]
    f32 = jnp.float32
    lane_grp = jax.lax.broadcasted_iota(jnp.int32, (blk, 128), 1) // 16
    uoff = uidx_ref[...] % 8
    ioff = iidx_ref[...] % 8
    mu = (lane_grp == uoff).astype(f32)
    mi = (lane_grp == ioff).astype(f32)
    l16 = jax.lax.broadcasted_iota(jnp.int32, (128, 16), 0) % 16
    k16 = jax.lax.broadcasted_iota(jnp.int32, (128, 16), 1)
    E = (l16 == k16).astype(f32)

    def extract(r_ref, m):
        return jnp.dot(r_ref[...] * m, E, preferred_element_type=f32)

    ue = extract(ru_mlp_ref, mu)
    ie = extract(ri_mlp_ref, mi)
    um = extract(ru_mf_ref, mu)
    im = extract(ri_mf_ref, mi)

    x = jnp.concatenate([ue, ie], axis=1)
    h = jnp.maximum(jnp.dot(x, W1_ref[...],
                            preferred_element_type=f32) + b1_ref[...], 0.0)
    h = jnp.maximum(jnp.dot(h, W2_ref[...],
                            preferred_element_type=f32) + b2_ref[...], 0.0)
    h = jnp.maximum(jnp.dot(h, W3_ref[...],
                            preferred_element_type=f32) + b3_ref[...], 0.0)
    mf = um * im
    v = jnp.concatenate([h, mf], axis=1)
    out_ref[...] = jnp.dot(v, Wa_ref[...],
                           preferred_element_type=f32) + ba_ref[...]


def kernel(user_indices, item_indices, emb_user_mlp, emb_item_mlp,
           emb_user_mf, emb_item_mf, W1, b1, W2, b2, W3, b3, Wa, ba):
    B = user_indices.shape[0]
    uidx = user_indices.astype(jnp.int32)
    iidx = item_indices.astype(jnp.int32)
    t_umlp = emb_user_mlp.reshape(-1, 128)
    t_imlp = emb_item_mlp.reshape(-1, 128)
    t_umf = emb_user_mf.reshape(-1, 128)
    t_imf = emb_item_mf.reshape(-1, 128)

    gather = _make_gather(B)
    ru_mlp, ri_mlp, ru_mf, ri_mf = gather(uidx, iidx, t_umlp, t_imlp,
                                          t_umf, t_imf)

    BLK = 2048
    grid = B // BLK
    row_spec = pl.BlockSpec((BLK, 128), lambda i: (i, 0))
    idx_spec = pl.BlockSpec((BLK, 1), lambda i: (i, 0))

    def w_spec(shape):
        return pl.BlockSpec(shape, lambda i: tuple(0 for _ in shape))

    logits = pl.pallas_call(
        _mlp_body,
        grid=(grid,),
        in_specs=[
            idx_spec, idx_spec, row_spec, row_spec, row_spec, row_spec,
            w_spec((32, 32)), w_spec((1, 32)), w_spec((32, 16)),
            w_spec((1, 16)), w_spec((16, 8)), w_spec((1, 8)),
            w_spec((24, 1)), w_spec((1, 1)),
        ],
        out_specs=pl.BlockSpec((BLK, 1), lambda i: (i, 0)),
        out_shape=jax.ShapeDtypeStruct((B, 1), jnp.float32),
    )(uidx.reshape(B, 1), iidx.reshape(B, 1), ru_mlp, ri_mlp, ru_mf, ri_mf,
      W1, b1.reshape(1, -1), W2, b2.reshape(1, -1), W3, b3.reshape(1, -1),
      Wa, ba.reshape(1, -1))
    return logits


# speedup vs baseline: 3.1051x; 3.1051x over previous
import functools

import jax
import jax.numpy as jnp
from jax import lax
from jax.experimental import pallas as pl
from jax.experimental.pallas import tpu as pltpu
from jax.experimental.pallas import tpu_sc as plsc

NC = 2
NS = 16
NW = NC * NS
D = 16
L = 16
W = 8192


def _depad_body(x_ref, o_ref):
    o_ref[...] = x_ref[...].reshape(D, W // 128, 128)


@functools.lru_cache(maxsize=None)
def _make_depad(NV):
    n_blk = -(-NV // W)
    return pl.pallas_call(
        _depad_body,
        grid=(n_blk,),
        in_specs=[pl.BlockSpec((D, W), lambda i: (0, i))],
        out_specs=pl.BlockSpec((D, W // 128, 128), lambda i: (0, i, 0)),
        out_shape=jax.ShapeDtypeStruct((D, n_blk * (W // 128), 128),
                                       jnp.float32),
    )


def _gather_body(idx_hbm, lines_hbm, out_hbm, idx_v, lane_v, off_v, stage,
                 blk, sem, *, b_per_w, m_lines):
    wid = lax.axis_index("s") * NC + lax.axis_index("c")
    base = wid * b_per_w
    pltpu.sync_copy(idx_hbm.at[pl.ds(base, b_per_w)], idx_v)
    for c in range(b_per_w // L):
        sl = pl.ds(c * L, L)
        i = idx_v[sl]
        lane_v[sl] = lax.bitwise_and(i, 127)
        idx_v[sl] = lax.shift_right_logical(i, 7)

    def chunk(c, carry):
        sl = pl.ds(c * L, L)
        base16 = idx_v[sl]
        for f in range(D):
            off_v[pl.ds(f * L, L)] = base16 + (f * m_lines)
        pltpu.async_copy(lines_hbm.at[off_v], stage, sem).wait()
        lanes16 = lane_v[sl]
        rows = lax.iota(jnp.int32, L)
        for f in range(D):
            vals = plsc.load_gather(stage, [rows + (f * L), lanes16])
            blk[f, sl] = vals
        return carry

    lax.fori_loop(0, b_per_w // L, chunk, 0)
    pltpu.sync_copy(blk, out_hbm.at[:, pl.ds(base, b_per_w)])


@functools.lru_cache(maxsize=None)
def _make_gather(B, m_lines):
    assert B % (8 * NW) == 0
    b_per_w = B // NW
    mesh = plsc.VectorSubcoreMesh(core_axis_name="c", subcore_axis_name="s",
                                  num_cores=NC, num_subcores=NS)
    f32 = jnp.float32
    return pl.kernel(
        functools.partial(_gather_body, b_per_w=b_per_w, m_lines=m_lines),
        out_type=jax.ShapeDtypeStruct((D, B), f32),
        mesh=mesh,
        scratch_types=[
            pltpu.VMEM((b_per_w,), jnp.int32),
            pltpu.VMEM((b_per_w,), jnp.int32),
            pltpu.VMEM((D * L,), jnp.int32),
            pltpu.VMEM((D * L, 128), f32),
            pltpu.VMEM((D, b_per_w), f32),
            pltpu.SemaphoreType.DMA,
        ],
        compiler_params=pltpu.CompilerParams(needs_layout_passes=False),
    )


def _mlp_body(ue_ref, ie_ref, um_ref, im_ref, W1_ref, b1_ref, W2_ref, b2_ref,
              W3_ref, b3_ref, Wa_ref, ba_ref, out_ref):
    f32 = jnp.float32
    dn0 = (((0,), (0,)), ((), ()))

    x = jnp.concatenate([ue_ref[...], ie_ref[...]], axis=0)
    h = jnp.maximum(lax.dot_general(W1_ref[...], x, dn0,
                                    preferred_element_type=f32) + b1_ref[...],
                    0.0)
    h = jnp.maximum(lax.dot_general(W2_ref[...], h, dn0,
                                    preferred_element_type=f32) + b2_ref[...],
                    0.0)
    h = jnp.maximum(lax.dot_general(W3_ref[...], h, dn0,
                                    preferred_element_type=f32) + b3_ref[...],
                    0.0)
    mf = um_ref[...] * im_ref[...]
    v = jnp.concatenate([h, mf], axis=0)
    out_ref[...] = lax.dot_general(Wa_ref[...], v, dn0,
                                   preferred_element_type=f32) + ba_ref[...]


def kernel(user_indices, item_indices, emb_user_mlp, emb_item_mlp,
           emb_user_mf, emb_item_mf, W1, b1, W2, b2, W3, b3, Wa, ba):
    B = user_indices.shape[0]
    NV = emb_user_mlp.shape[0]
    uidx = user_indices.astype(jnp.int32)
    iidx = item_indices.astype(jnp.int32)

    depad = _make_depad(NV)
    m_lines = (-(-NV // W)) * (W // 128)
    gather = _make_gather(B, m_lines)

    acts = []
    for table, idx in ((emb_user_mlp, uidx), (emb_item_mlp, iidx),
                       (emb_user_mf, uidx), (emb_item_mf, iidx)):
        lines3 = depad(table.T)
        lines = lines3.reshape(D * m_lines, 128)
        acts.append(gather(idx, lines))
    ue, ie, um, im = acts

    BLK = 4096
    grid = B // BLK
    act_spec = pl.BlockSpec((D, BLK), lambda i: (0, i))

    def w_spec(shape):
        return pl.BlockSpec(shape, lambda i: tuple(0 for _ in shape))

    out = pl.pallas_call(
        _mlp_body,
        grid=(grid,),
        in_specs=[
            act_spec, act_spec, act_spec, act_spec,
            w_spec((32, 32)), w_spec((32, 1)), w_spec((32, 16)),
            w_spec((16, 1)), w_spec((16, 8)), w_spec((8, 1)),
            w_spec((24, 1)), w_spec((1, 1)),
        ],
        out_specs=pl.BlockSpec((1, BLK), lambda i: (0, i)),
        out_shape=jax.ShapeDtypeStruct((1, B), jnp.float32),
    )(ue, ie, um, im,
      W1, b1.reshape(-1, 1), W2, b2.reshape(-1, 1), W3, b3.reshape(-1, 1),
      Wa, ba.reshape(-1, 1))
    return out.reshape(B, 1)
